# Initial kernel scaffold; baseline (speedup 1.0000x reference)
import jax
import jax.numpy as jnp
from jax.experimental import pallas as pl


def kernel(xyz, W0a, b0a, W0b, b0b, W1, b1, W2, b2, Wd1, bd1, Wd2, bd2, Wg1, bg1, Wg2, bg2, Wq, Wk, Wv):
    raise NotImplementedError("write your pallas kernel here")


# speedup vs baseline: 9.0594x; 9.0594x over previous
import functools

import jax
import jax.numpy as jnp
from jax import lax
from jax.experimental import pallas as pl
from jax.experimental.pallas import tpu as pltpu
from jax.experimental.pallas import tpu_sc as plsc

_HI = lax.Precision.HIGHEST

_B, _N, _INF, _TF, _D, _K = 2, 1024, 64, 128, 256, 16
_TQ = 256
_TM = 128
_TW = 640


def _wprep_body(W1, b1, Wq, Wk, Wv, Wg1, Wd2, bd2, bg1,
                Aq, cq, Ak, ck, Av, cv, Wd2g):
    w1 = W1[:]
    g1 = Wg1[:]
    Aq[:] = jnp.dot(jnp.dot(w1, Wq[:], precision=_HI), g1, precision=_HI)
    Ak[:] = jnp.dot(jnp.dot(w1, Wk[:], precision=_HI), g1, precision=_HI)
    Av[:] = jnp.dot(w1, Wv[:], precision=_HI)
    b1v = b1[:]
    cq[:] = (jnp.dot(jnp.dot(b1v, Wq[:], precision=_HI), g1, precision=_HI)
             + jnp.dot(bd2[:], g1, precision=_HI) + bg1[:])
    ck[:] = jnp.dot(jnp.dot(b1v, Wk[:], precision=_HI), g1, precision=_HI)
    cv[:] = jnp.dot(b1v, Wv[:], precision=_HI)
    Wd2g[:] = jnp.dot(Wd2[:], g1, precision=_HI)


def _wprep(W1, b1, Wq, Wk, Wv, Wg1, Wd2, bd2, bg1):
    f32 = jnp.float32
    outs = (
        jax.ShapeDtypeStruct((_TF, _D), f32),
        jax.ShapeDtypeStruct((1, _D), f32),
        jax.ShapeDtypeStruct((_TF, _D), f32),
        jax.ShapeDtypeStruct((1, _D), f32),
        jax.ShapeDtypeStruct((_TF, _D), f32),
        jax.ShapeDtypeStruct((1, _D), f32),
        jax.ShapeDtypeStruct((_D, _D), f32),
    )
    return pl.pallas_call(_wprep_body, out_shape=outs)(
        W1, b1.reshape(1, _D), Wq, Wk, Wv, Wg1, Wd2,
        bd2.reshape(1, _D), bg1.reshape(1, _D))


def _stage1_body(xq_ref, xf_ref, W0a, b0a, W0b, b0b,
                 Aq, cq, Ak, ck, Av, cv,
                 table_ref, qg_ref, pre_ref, idx_ref):
    b = pl.program_id(0)
    xq = xq_ref[0]
    xf = xf_ref[0]

    f1 = jnp.maximum(jnp.dot(xq, W0a[:]) + b0a[:], 0.0)
    feats = jnp.dot(f1, W0b[:]) + b0b[:]
    pre_ref[:] = feats
    qg_ref[:] = jnp.dot(feats, Aq[:]) + cq[:]
    kg1 = jnp.dot(feats, Ak[:]) + ck[:]
    v = jnp.dot(feats, Av[:]) + cv[:]
    table_ref[:] = jnp.concatenate(
        [xq, jnp.zeros((_TQ, _INF), jnp.float32), kg1, v], axis=1)

    d = -2.0 * lax.dot_general(xq, xf, (((1,), (1,)), ((), ())))
    d = d + jnp.sum(xq * xq, axis=1, keepdims=True)
    d = d + jnp.sum(xf * xf, axis=1)[None, :]

    col = lax.broadcasted_iota(jnp.int32, (_TQ, _N), 1)
    big = jnp.float32(3.0e38)
    vals = d
    sels = []
    for _ in range(_K):
        m = jnp.min(vals, axis=1, keepdims=True)
        cand = jnp.where(vals <= m, col, jnp.int32(_N))
        sel = jnp.min(cand, axis=1, keepdims=True)
        sels.append(sel)
        vals = jnp.where(col == sel, big, vals)
    idx_ref[:] = jnp.concatenate(sels, axis=1) + b * _N


def _stage1(xyz, W0a, b0a, W0b, b0b, Aq, cq, Ak, ck, Av, cv):
    f32 = jnp.float32
    nt = _N // _TQ
    grid = (_B, nt)
    row = lambda b, t: (b * nt + t, 0)
    full = lambda b, t: (0, 0)
    in_specs = [
        pl.BlockSpec((1, _TQ, _INF), lambda b, t: (b, t, 0)),
        pl.BlockSpec((1, _N, _INF), lambda b, t: (b, 0, 0)),
        pl.BlockSpec((_INF, _TF), full),
        pl.BlockSpec((1, _TF), full),
        pl.BlockSpec((_TF, _TF), full),
        pl.BlockSpec((1, _TF), full),
        pl.BlockSpec((_TF, _D), full),
        pl.BlockSpec((1, _D), full),
        pl.BlockSpec((_TF, _D), full),
        pl.BlockSpec((1, _D), full),
        pl.BlockSpec((_TF, _D), full),
        pl.BlockSpec((1, _D), full),
    ]
    out_specs = [
        pl.BlockSpec((_TQ, _TW), row),
        pl.BlockSpec((_TQ, _D), row),
        pl.BlockSpec((_TQ, _TF), row),
        pl.BlockSpec((_TQ, _K), row),
    ]
    outs = (
        jax.ShapeDtypeStruct((_B * _N, _TW), f32),
        jax.ShapeDtypeStruct((_B * _N, _D), f32),
        jax.ShapeDtypeStruct((_B * _N, _TF), f32),
        jax.ShapeDtypeStruct((_B * _N, _K), jnp.int32),
    )
    return pl.pallas_call(
        _stage1_body, grid=grid, in_specs=in_specs, out_specs=out_specs,
        out_shape=outs,
    )(xyz, xyz, W0a, b0a.reshape(1, _TF), W0b, b0b.reshape(1, _TF),
      Aq, cq, Ak, ck, Av, cv)


def _sc_gather(table, idx_flat):
    tot = idx_flat.shape[0]
    nw = 32
    per_w = tot // nw
    ch = 128
    n_ch = per_w // ch

    mesh = plsc.VectorSubcoreMesh(core_axis_name="c", subcore_axis_name="s")

    @functools.partial(
        pl.kernel, mesh=mesh,
        out_type=jax.ShapeDtypeStruct((tot, _TW), jnp.float32),
        scratch_types=[
            pltpu.VMEM((ch,), jnp.int32),
            pltpu.VMEM((ch, _TW), jnp.float32),
            pltpu.SemaphoreType.DMA,
        ],
    )
    def gather_kernel(table_hbm, idx_hbm, out_hbm, idx_v, rows_v, sem):
        wid = lax.axis_index("s") * 2 + lax.axis_index("c")
        base = wid * per_w
        for c in range(n_ch):
            off = base + c * ch
            pltpu.sync_copy(idx_hbm.at[pl.ds(off, ch)], idx_v)
            pltpu.async_copy(table_hbm.at[idx_v], rows_v, sem).wait()
            pltpu.sync_copy(rows_v, out_hbm.at[pl.ds(off, ch)])

    return gather_kernel(table, idx_flat)


def _stage3_body(g_ref, xyz_ref, qg_ref, pre_ref,
                 Wd1, bd1, Wd2, bd2, Wd2g, Wg2, bg2, W2, b2,
                 attn_ref, res_ref):
    g = g_ref[:]
    xq = xyz_ref[:]
    pos = xq[:, None, :] - g[:, :, 0:_INF]
    pos2 = pos.reshape(_TM * _K, _INF)
    h = jnp.maximum(jnp.dot(pos2, Wd1[:]) + bd1[:], 0.0)
    pe = jnp.dot(h, Wd2[:]) + bd2[:]
    a3 = jnp.dot(h, Wd2g[:])

    kg2 = g[:, :, 2 * _INF:2 * _INF + _D].reshape(_TM * _K, _D)
    v2 = g[:, :, 2 * _INF + _D:].reshape(_TM * _K, _D)
    qg2 = jnp.broadcast_to(qg_ref[:][:, None, :], (_TM, _K, _D))
    qg2 = qg2.reshape(_TM * _K, _D)

    l1 = jnp.maximum(qg2 - kg2 + a3, 0.0)
    logits = (jnp.dot(l1, Wg2[:]) + bg2[:]) * jnp.float32(1.0 / 16.0)

    lg3 = logits.reshape(_TM, _K, _D)
    m = jnp.max(lg3, axis=1, keepdims=True)
    e = jnp.exp(lg3 - m)
    s = jnp.sum(e, axis=1, keepdims=True)
    attn = e / s
    attn_ref[:] = attn

    pe3 = pe.reshape(_TM, _K, _D)
    v3 = v2.reshape(_TM, _K, _D)
    wsum = jnp.sum(attn * (v3 + pe3), axis=1)
    res_ref[:] = jnp.dot(wsum, W2[:]) + b2[:] + pre_ref[:]


def _stage3(g3, xyzf, qg, pre, Wd1, bd1, Wd2, bd2, Wd2g, Wg2, bg2, W2, b2):
    f32 = jnp.float32
    grid = (_B * _N // _TM,)
    row = lambda t: (t, 0)
    row3 = lambda t: (t, 0, 0)
    full = lambda t: (0, 0)
    in_specs = [
        pl.BlockSpec((_TM, _K, _TW), row3),
        pl.BlockSpec((_TM, _INF), row),
        pl.BlockSpec((_TM, _D), row),
        pl.BlockSpec((_TM, _TF), row),
        pl.BlockSpec((_INF, _D), full),
        pl.BlockSpec((1, _D), full),
        pl.BlockSpec((_D, _D), full),
        pl.BlockSpec((1, _D), full),
        pl.BlockSpec((_D, _D), full),
        pl.BlockSpec((_D, _D), full),
        pl.BlockSpec((1, _D), full),
        pl.BlockSpec((_D, _TF), full),
        pl.BlockSpec((1, _TF), full),
    ]
    out_specs = [
        pl.BlockSpec((_TM, _K, _D), row3),
        pl.BlockSpec((_TM, _TF), row),
    ]
    outs = (
        jax.ShapeDtypeStruct((_B * _N, _K, _D), f32),
        jax.ShapeDtypeStruct((_B * _N, _TF), f32),
    )
    return pl.pallas_call(
        _stage3_body, grid=grid, in_specs=in_specs, out_specs=out_specs,
        out_shape=outs,
    )(g3, xyzf, qg, pre, Wd1, bd1.reshape(1, _D), Wd2, bd2.reshape(1, _D),
      Wd2g, Wg2, bg2.reshape(1, _D), W2, b2.reshape(1, _TF))


def kernel(xyz, W0a, b0a, W0b, b0b, W1, b1, W2, b2, Wd1, bd1, Wd2, bd2,
           Wg1, bg1, Wg2, bg2, Wq, Wk, Wv):
    assert xyz.shape == (_B, _N, _INF)
    Aq, cq, Ak, ck, Av, cv, Wd2g = _wprep(W1, b1, Wq, Wk, Wv, Wg1, Wd2,
                                          bd2, bg1)
    table, qg, pre, idx = _stage1(xyz, W0a, b0a, W0b, b0b,
                                  Aq, cq, Ak, ck, Av, cv)
    g = _sc_gather(table, idx.reshape(_B * _N * _K))
    attn, res = _stage3(g.reshape(_B * _N, _K, _TW), xyz.reshape(_B * _N, _INF),
                        qg, pre, Wd1, bd1, Wd2, bd2, Wd2g, Wg2, bg2, W2, b2)
    return (res.reshape(_B, _N, _TF), attn.reshape(_B, _N, _K, _D))


# speedup vs baseline: 10.1870x; 1.1245x over previous
import functools

import jax
import jax.numpy as jnp
from jax import lax
from jax.experimental import pallas as pl
from jax.experimental.pallas import tpu as pltpu
from jax.experimental.pallas import tpu_sc as plsc

_HI = lax.Precision.HIGHEST

_B, _N, _INF, _TF, _D, _K = 2, 1024, 64, 128, 256, 16
_TQ = 256
_TM = 128
_TW = 256
_NH = 2


def _wprep_body(W1, b1, Wq, Wk, Wv, Wg1, Wd2, bd2, bg1,
                Aq, cq, Ak, ck, Av, cv, Wd2g):
    w1 = W1[:]
    g1 = Wg1[:]
    Aq[:] = jnp.dot(jnp.dot(w1, Wq[:], precision=_HI), g1, precision=_HI)
    Ak[:] = jnp.dot(jnp.dot(w1, Wk[:], precision=_HI), g1, precision=_HI)
    Av[:] = jnp.dot(w1, Wv[:], precision=_HI)
    b1v = b1[:]
    cq[:] = (jnp.dot(jnp.dot(b1v, Wq[:], precision=_HI), g1, precision=_HI)
             + jnp.dot(bd2[:], g1, precision=_HI) + bg1[:])
    ck[:] = jnp.dot(jnp.dot(b1v, Wk[:], precision=_HI), g1, precision=_HI)
    cv[:] = jnp.dot(b1v, Wv[:], precision=_HI)
    Wd2g[:] = jnp.dot(Wd2[:], g1, precision=_HI)


def _wprep(W1, b1, Wq, Wk, Wv, Wg1, Wd2, bd2, bg1):
    f32 = jnp.float32
    outs = (
        jax.ShapeDtypeStruct((_TF, _D), f32),
        jax.ShapeDtypeStruct((1, _D), f32),
        jax.ShapeDtypeStruct((_TF, _D), f32),
        jax.ShapeDtypeStruct((1, _D), f32),
        jax.ShapeDtypeStruct((_TF, _D), f32),
        jax.ShapeDtypeStruct((1, _D), f32),
        jax.ShapeDtypeStruct((_D, _D), f32),
    )
    return pl.pallas_call(_wprep_body, out_shape=outs)(
        W1, b1.reshape(1, _D), Wq, Wk, Wv, Wg1, Wd2,
        bd2.reshape(1, _D), bg1.reshape(1, _D))


def _stage1_body(xq_ref, xf_ref, W0a, b0a, W0b, b0b, table_ref, idx_ref):
    b = pl.program_id(0)
    xq = xq_ref[0]
    xf = xf_ref[0]

    f1 = jnp.maximum(jnp.dot(xq, W0a[:]) + b0a[:], 0.0)
    feats = jnp.dot(f1, W0b[:]) + b0b[:]
    table_ref[:] = jnp.concatenate(
        [xq, jnp.zeros((_TQ, _INF), jnp.float32), feats], axis=1)

    d = -2.0 * lax.dot_general(xq, xf, (((1,), (1,)), ((), ())))
    d = d + jnp.sum(xq * xq, axis=1, keepdims=True)
    d = d + jnp.sum(xf * xf, axis=1)[None, :]

    colf = lax.broadcasted_iota(jnp.int32, (_TQ, _N), 1).astype(jnp.float32)
    big = jnp.float32(3.0e38)
    vals = d
    sels = []
    for _ in range(_K):
        m = jnp.min(vals, axis=1, keepdims=True)
        cand = jnp.where(vals <= m, colf, jnp.float32(_N))
        sel = jnp.min(cand, axis=1, keepdims=True)
        sels.append(sel)
        vals = jnp.where(colf == sel, big, vals)
    idx_ref[:] = jnp.concatenate(sels, axis=1).astype(jnp.int32) + b * _N


def _stage1(xyz, W0a, b0a, W0b, b0b):
    f32 = jnp.float32
    nt = _N // _TQ
    grid = (_B, nt)
    row = lambda b, t: (b * nt + t, 0)
    full = lambda b, t: (0, 0)
    in_specs = [
        pl.BlockSpec((1, _TQ, _INF), lambda b, t: (b, t, 0)),
        pl.BlockSpec((1, _N, _INF), lambda b, t: (b, 0, 0)),
        pl.BlockSpec((_INF, _TF), full),
        pl.BlockSpec((1, _TF), full),
        pl.BlockSpec((_TF, _TF), full),
        pl.BlockSpec((1, _TF), full),
    ]
    out_specs = [
        pl.BlockSpec((_TQ, _TW), row),
        pl.BlockSpec((_TQ, _K), row),
    ]
    outs = (
        jax.ShapeDtypeStruct((_B * _N, _TW), f32),
        jax.ShapeDtypeStruct((_B * _N, _K), jnp.int32),
    )
    return pl.pallas_call(
        _stage1_body, grid=grid, in_specs=in_specs, out_specs=out_specs,
        out_shape=outs,
    )(xyz, xyz, W0a, b0a.reshape(1, _TF), W0b, b0b.reshape(1, _TF))


def _sc_gather(table, idx_flat):
    tot = idx_flat.shape[0]
    nw = 32
    per_w = tot // nw
    ch = 128
    n_ch = per_w // ch

    mesh = plsc.VectorSubcoreMesh(core_axis_name="c", subcore_axis_name="s")

    @functools.partial(
        pl.kernel, mesh=mesh,
        out_type=jax.ShapeDtypeStruct((tot, _TW), jnp.float32),
        scratch_types=[
            pltpu.VMEM((ch,), jnp.int32),
            pltpu.VMEM((ch, _TW), jnp.float32),
            pltpu.SemaphoreType.DMA,
        ],
    )
    def gather_kernel(table_hbm, idx_hbm, out_hbm, idx_v, rows_v, sem):
        wid = lax.axis_index("s") * 2 + lax.axis_index("c")
        base = wid * per_w
        for c in range(n_ch):
            off = base + c * ch
            pltpu.sync_copy(idx_hbm.at[pl.ds(off, ch)], idx_v)
            pltpu.async_copy(table_hbm.at[idx_v], rows_v, sem).wait()
            pltpu.sync_copy(rows_v, out_hbm.at[pl.ds(off, ch)])

    return gather_kernel(table, idx_flat)


def _stage3_body(g_ref, xyz_ref, pre_ref,
                 Aq, cq, Ak, ck, Av, cv,
                 Wd1, bd1, Wd2, bd2, Wd2g, Wg2, bg2, W2, b2,
                 attn_ref, res_ref):
    g = g_ref[:]
    xq = xyz_ref[:]
    fq = pre_ref[:]
    pos = xq[:, None, :] - g[:, :, 0:_INF]
    pos2 = pos.reshape(_TM * _K, _INF)
    h = jnp.maximum(jnp.dot(pos2, Wd1[:]) + bd1[:], 0.0)
    pe = jnp.dot(h, Wd2[:]) + bd2[:]
    a3 = jnp.dot(h, Wd2g[:])

    f2 = g[:, :, _TF:].reshape(_TM * _K, _TF)
    kg2 = jnp.dot(f2, Ak[:]) + ck[:]
    v2 = jnp.dot(f2, Av[:]) + cv[:]
    qg = jnp.dot(fq, Aq[:]) + cq[:]
    qg2 = jnp.broadcast_to(qg[:, None, :], (_TM, _K, _D))
    qg2 = qg2.reshape(_TM * _K, _D)

    l1 = jnp.maximum(qg2 - kg2 + a3, 0.0)
    logits = (jnp.dot(l1, Wg2[:]) + bg2[:]) * jnp.float32(1.0 / 16.0)

    lg3 = logits.reshape(_TM, _K, _D)
    m = jnp.max(lg3, axis=1, keepdims=True)
    e = jnp.exp(lg3 - m)
    s = jnp.sum(e, axis=1, keepdims=True)
    attn = e / s
    attn_ref[:] = attn

    pe3 = pe.reshape(_TM, _K, _D)
    v3 = v2.reshape(_TM, _K, _D)
    wsum = jnp.sum(attn * (v3 + pe3), axis=1)
    res_ref[:] = jnp.dot(wsum, W2[:]) + b2[:] + fq


def _stage3(half, g3, xyzf, table, Aq, cq, Ak, ck, Av, cv,
            Wd1, bd1, Wd2, bd2, Wd2g, Wg2, bg2, W2, b2):
    f32 = jnp.float32
    bnh = _B * _N // _NH
    nt = bnh // _TM
    grid = (nt,)
    row = lambda t: (t, 0)
    row3 = lambda t: (t, 0, 0)
    hrow = lambda t: (half * nt + t, 0)
    hrow_f = lambda t: (half * nt + t, 1)
    full = lambda t: (0, 0)
    in_specs = [
        pl.BlockSpec((_TM, _K, _TW), row3),
        pl.BlockSpec((_TM, _INF), hrow),
        pl.BlockSpec((_TM, _TF), hrow_f),
        pl.BlockSpec((_TF, _D), full),
        pl.BlockSpec((1, _D), full),
        pl.BlockSpec((_TF, _D), full),
        pl.BlockSpec((1, _D), full),
        pl.BlockSpec((_TF, _D), full),
        pl.BlockSpec((1, _D), full),
        pl.BlockSpec((_INF, _D), full),
        pl.BlockSpec((1, _D), full),
        pl.BlockSpec((_D, _D), full),
        pl.BlockSpec((1, _D), full),
        pl.BlockSpec((_D, _D), full),
        pl.BlockSpec((_D, _D), full),
        pl.BlockSpec((1, _D), full),
        pl.BlockSpec((_D, _TF), full),
        pl.BlockSpec((1, _TF), full),
    ]
    out_specs = [
        pl.BlockSpec((_TM, _K, _D), row3),
        pl.BlockSpec((_TM, _TF), row),
    ]
    outs = (
        jax.ShapeDtypeStruct((bnh, _K, _D), f32),
        jax.ShapeDtypeStruct((bnh, _TF), f32),
    )
    return pl.pallas_call(
        _stage3_body, grid=grid, in_specs=in_specs, out_specs=out_specs,
        out_shape=outs,
    )(g3, xyzf, table, Aq, cq, Ak, ck, Av, cv,
      Wd1, bd1.reshape(1, _D), Wd2, bd2.reshape(1, _D), Wd2g,
      Wg2, bg2.reshape(1, _D), W2, b2.reshape(1, _TF))


def kernel(xyz, W0a, b0a, W0b, b0b, W1, b1, W2, b2, Wd1, bd1, Wd2, bd2,
           Wg1, bg1, Wg2, bg2, Wq, Wk, Wv):
    assert xyz.shape == (_B, _N, _INF)
    Aq, cq, Ak, ck, Av, cv, Wd2g = _wprep(W1, b1, Wq, Wk, Wv, Wg1, Wd2,
                                          bd2, bg1)
    table, idx = _stage1(xyz, W0a, b0a, W0b, b0b)
    idxf = idx.reshape(_B * _N * _K)
    xyzf = xyz.reshape(_B * _N, _INF)
    hk = _B * _N * _K // _NH
    hr = _B * _N // _NH
    attn_h, res_h = [], []
    for half in range(_NH):
        g = _sc_gather(table, lax.slice(idxf, (half * hk,),
                                        ((half + 1) * hk,)))
        a, r = _stage3(half, g.reshape(hr, _K, _TW), xyzf, table,
                       Aq, cq, Ak, ck, Av, cv,
                       Wd1, bd1, Wd2, bd2, Wd2g, Wg2, bg2, W2, b2)
        attn_h.append(a)
        res_h.append(r)
    res = jnp.concatenate(res_h, axis=0).reshape(_B, _N, _TF)
    attn = jnp.concatenate(attn_h, axis=0).reshape(_B, _N, _K, _D)
    return (res, attn)


# speedup vs baseline: 10.2711x; 1.0083x over previous
import functools

import jax
import jax.numpy as jnp
from jax import lax
from jax.experimental import pallas as pl
from jax.experimental.pallas import tpu as pltpu
from jax.experimental.pallas import tpu_sc as plsc

_HI = lax.Precision.HIGHEST

_B, _N, _INF, _TF, _D, _K = 2, 1024, 64, 128, 256, 16
_TQ = 256
_TM = 128
_TW = 256
_NH = 2


def _wprep_body(W1, b1, Wq, Wk, Wv, Wg1, Wd2, bd2, bg1,
                Aq, cq, Ak, ck, Av, cv, Wd2g):
    w1 = W1[:]
    g1 = Wg1[:]
    Aq[:] = jnp.dot(jnp.dot(w1, Wq[:], precision=_HI), g1, precision=_HI)
    Ak[:] = jnp.dot(jnp.dot(w1, Wk[:], precision=_HI), g1, precision=_HI)
    Av[:] = jnp.dot(w1, Wv[:], precision=_HI)
    b1v = b1[:]
    cq[:] = (jnp.dot(jnp.dot(b1v, Wq[:], precision=_HI), g1, precision=_HI)
             + jnp.dot(bd2[:], g1, precision=_HI) + bg1[:])
    ck[:] = jnp.dot(jnp.dot(b1v, Wk[:], precision=_HI), g1, precision=_HI)
    cv[:] = jnp.dot(b1v, Wv[:], precision=_HI)
    Wd2g[:] = jnp.dot(Wd2[:], g1, precision=_HI)


def _wprep(W1, b1, Wq, Wk, Wv, Wg1, Wd2, bd2, bg1):
    f32 = jnp.float32
    outs = (
        jax.ShapeDtypeStruct((_TF, _D), f32),
        jax.ShapeDtypeStruct((1, _D), f32),
        jax.ShapeDtypeStruct((_TF, _D), f32),
        jax.ShapeDtypeStruct((1, _D), f32),
        jax.ShapeDtypeStruct((_TF, _D), f32),
        jax.ShapeDtypeStruct((1, _D), f32),
        jax.ShapeDtypeStruct((_D, _D), f32),
    )
    return pl.pallas_call(_wprep_body, out_shape=outs)(
        W1, b1.reshape(1, _D), Wq, Wk, Wv, Wg1, Wd2,
        bd2.reshape(1, _D), bg1.reshape(1, _D))


def _stage1_body(xq_ref, xf_ref, W0a, b0a, W0b, b0b, table_ref, idx_ref):
    b = pl.program_id(0)
    xq = xq_ref[0]
    xf = xf_ref[0]

    f1 = jnp.maximum(jnp.dot(xq, W0a[:]) + b0a[:], 0.0)
    feats = jnp.dot(f1, W0b[:]) + b0b[:]
    table_ref[:] = jnp.concatenate(
        [xq, jnp.zeros((_TQ, _INF), jnp.float32), feats], axis=1)

    d = -2.0 * lax.dot_general(xq, xf, (((1,), (1,)), ((), ())))
    d = d + jnp.sum(xq * xq, axis=1, keepdims=True)
    d = d + jnp.sum(xf * xf, axis=1)[None, :]

    colf = lax.broadcasted_iota(jnp.int32, (_TQ, _N), 1).astype(jnp.float32)
    big = jnp.float32(3.0e38)
    vals = d
    sels = []
    for _ in range(_K):
        m = jnp.min(vals, axis=1, keepdims=True)
        cand = jnp.where(vals <= m, colf, jnp.float32(_N))
        sel = jnp.min(cand, axis=1, keepdims=True)
        sels.append(sel)
        vals = jnp.where(colf == sel, big, vals)
    idx_ref[:] = jnp.concatenate(sels, axis=1).astype(jnp.int32) + b * _N


def _stage1(xyz, W0a, b0a, W0b, b0b):
    f32 = jnp.float32
    nt = _N // _TQ
    grid = (_B, nt)
    row = lambda b, t: (b * nt + t, 0)
    full = lambda b, t: (0, 0)
    in_specs = [
        pl.BlockSpec((1, _TQ, _INF), lambda b, t: (b, t, 0)),
        pl.BlockSpec((1, _N, _INF), lambda b, t: (b, 0, 0)),
        pl.BlockSpec((_INF, _TF), full),
        pl.BlockSpec((1, _TF), full),
        pl.BlockSpec((_TF, _TF), full),
        pl.BlockSpec((1, _TF), full),
    ]
    out_specs = [
        pl.BlockSpec((_TQ, _TW), row),
        pl.BlockSpec((_TQ, _K), row),
    ]
    outs = (
        jax.ShapeDtypeStruct((_B * _N, _TW), f32),
        jax.ShapeDtypeStruct((_B * _N, _K), jnp.int32),
    )
    return pl.pallas_call(
        _stage1_body, grid=grid, in_specs=in_specs, out_specs=out_specs,
        out_shape=outs,
    )(xyz, xyz, W0a, b0a.reshape(1, _TF), W0b, b0b.reshape(1, _TF))


def _sc_gather(table, idx_flat):
    tot = idx_flat.shape[0]
    nw = 32
    per_w = tot // nw
    ch = 128
    n_ch = per_w // ch

    mesh = plsc.VectorSubcoreMesh(core_axis_name="c", subcore_axis_name="s")

    @functools.partial(
        pl.kernel, mesh=mesh,
        out_type=jax.ShapeDtypeStruct((tot, _TW), jnp.float32),
        scratch_types=[
            pltpu.VMEM((per_w,), jnp.int32),
            pltpu.VMEM((ch, _TW), jnp.float32),
            pltpu.VMEM((ch, _TW), jnp.float32),
            pltpu.SemaphoreType.DMA,
            pltpu.SemaphoreType.DMA,
        ],
    )
    def gather_kernel(table_hbm, idx_hbm, out_hbm, idx_v, rows_a, rows_b,
                      sem_a, sem_b):
        wid = lax.axis_index("s") * 2 + lax.axis_index("c")
        base = wid * per_w
        pltpu.sync_copy(idx_hbm.at[pl.ds(base, per_w)], idx_v)
        bufs = (rows_a, rows_b)
        sems = (sem_a, sem_b)
        cps = [None, None]
        for c in range(n_ch + 1):
            if c < n_ch:
                p = c % 2
                cps[p] = pltpu.async_copy(
                    table_hbm.at[idx_v.at[pl.ds(c * ch, ch)]],
                    bufs[p], sems[p])
            if c >= 1:
                p = (c - 1) % 2
                cps[p].wait()
                pltpu.sync_copy(bufs[p],
                                out_hbm.at[pl.ds(base + (c - 1) * ch, ch)])

    return gather_kernel(table, idx_flat)


def _stage3_body(g_ref, xyz_ref, pre_ref,
                 Aq, cq, Ak, ck, Av, cv,
                 Wd1, bd1, Wd2, bd2, Wd2g, Wg2, bg2, W2, b2,
                 attn_ref, res_ref):
    g = g_ref[:]
    xq = xyz_ref[:]
    fq = pre_ref[:]
    pos = xq[:, None, :] - g[:, :, 0:_INF]
    pos2 = pos.reshape(_TM * _K, _INF)
    h = jnp.maximum(jnp.dot(pos2, Wd1[:]) + bd1[:], 0.0)
    pe = jnp.dot(h, Wd2[:]) + bd2[:]
    a3 = jnp.dot(h, Wd2g[:])

    f2 = g[:, :, _TF:].reshape(_TM * _K, _TF)
    kg2 = jnp.dot(f2, Ak[:]) + ck[:]
    v2 = jnp.dot(f2, Av[:]) + cv[:]
    qg = jnp.dot(fq, Aq[:]) + cq[:]
    qg2 = jnp.broadcast_to(qg[:, None, :], (_TM, _K, _D))
    qg2 = qg2.reshape(_TM * _K, _D)

    l1 = jnp.maximum(qg2 - kg2 + a3, 0.0)
    logits = (jnp.dot(l1, Wg2[:]) + bg2[:]) * jnp.float32(1.0 / 16.0)

    lg3 = logits.reshape(_TM, _K, _D)
    m = jnp.max(lg3, axis=1, keepdims=True)
    e = jnp.exp(lg3 - m)
    s = jnp.sum(e, axis=1, keepdims=True)
    attn = e / s
    attn_ref[:] = attn

    pe3 = pe.reshape(_TM, _K, _D)
    v3 = v2.reshape(_TM, _K, _D)
    wsum = jnp.sum(attn * (v3 + pe3), axis=1)
    res_ref[:] = jnp.dot(wsum, W2[:]) + b2[:] + fq


def _stage3(half, g3, xyzf, table, Aq, cq, Ak, ck, Av, cv,
            Wd1, bd1, Wd2, bd2, Wd2g, Wg2, bg2, W2, b2):
    f32 = jnp.float32
    bnh = _B * _N // _NH
    nt = bnh // _TM
    grid = (nt,)
    row = lambda t: (t, 0)
    row3 = lambda t: (t, 0, 0)
    hrow = lambda t: (half * nt + t, 0)
    hrow_f = lambda t: (half * nt + t, 1)
    full = lambda t: (0, 0)
    in_specs = [
        pl.BlockSpec((_TM, _K, _TW), row3),
        pl.BlockSpec((_TM, _INF), hrow),
        pl.BlockSpec((_TM, _TF), hrow_f),
        pl.BlockSpec((_TF, _D), full),
        pl.BlockSpec((1, _D), full),
        pl.BlockSpec((_TF, _D), full),
        pl.BlockSpec((1, _D), full),
        pl.BlockSpec((_TF, _D), full),
        pl.BlockSpec((1, _D), full),
        pl.BlockSpec((_INF, _D), full),
        pl.BlockSpec((1, _D), full),
        pl.BlockSpec((_D, _D), full),
        pl.BlockSpec((1, _D), full),
        pl.BlockSpec((_D, _D), full),
        pl.BlockSpec((_D, _D), full),
        pl.BlockSpec((1, _D), full),
        pl.BlockSpec((_D, _TF), full),
        pl.BlockSpec((1, _TF), full),
    ]
    out_specs = [
        pl.BlockSpec((_TM, _K, _D), row3),
        pl.BlockSpec((_TM, _TF), row),
    ]
    outs = (
        jax.ShapeDtypeStruct((bnh, _K, _D), f32),
        jax.ShapeDtypeStruct((bnh, _TF), f32),
    )
    return pl.pallas_call(
        _stage3_body, grid=grid, in_specs=in_specs, out_specs=out_specs,
        out_shape=outs,
    )(g3, xyzf, table, Aq, cq, Ak, ck, Av, cv,
      Wd1, bd1.reshape(1, _D), Wd2, bd2.reshape(1, _D), Wd2g,
      Wg2, bg2.reshape(1, _D), W2, b2.reshape(1, _TF))


def kernel(xyz, W0a, b0a, W0b, b0b, W1, b1, W2, b2, Wd1, bd1, Wd2, bd2,
           Wg1, bg1, Wg2, bg2, Wq, Wk, Wv):
    assert xyz.shape == (_B, _N, _INF)
    Aq, cq, Ak, ck, Av, cv, Wd2g = _wprep(W1, b1, Wq, Wk, Wv, Wg1, Wd2,
                                          bd2, bg1)
    table, idx = _stage1(xyz, W0a, b0a, W0b, b0b)
    idxf = idx.reshape(_B * _N * _K)
    xyzf = xyz.reshape(_B * _N, _INF)
    hk = _B * _N * _K // _NH
    hr = _B * _N // _NH
    attn_h, res_h = [], []
    for half in range(_NH):
        g = _sc_gather(table, lax.slice(idxf, (half * hk,),
                                        ((half + 1) * hk,)))
        a, r = _stage3(half, g.reshape(hr, _K, _TW), xyzf, table,
                       Aq, cq, Ak, ck, Av, cv,
                       Wd1, bd1, Wd2, bd2, Wd2g, Wg2, bg2, W2, b2)
        attn_h.append(a)
        res_h.append(r)
    res = jnp.concatenate(res_h, axis=0).reshape(_B, _N, _TF)
    attn = jnp.concatenate(attn_h, axis=0).reshape(_B, _N, _K, _D)
    return (res, attn)


# speedup vs baseline: 12.5980x; 1.2265x over previous
import functools

import jax
import jax.numpy as jnp
from jax import lax
from jax.experimental import pallas as pl
from jax.experimental.pallas import tpu as pltpu
from jax.experimental.pallas import tpu_sc as plsc

_HI = lax.Precision.HIGHEST

_B, _N, _INF, _TF, _D, _K = 2, 1024, 64, 128, 256, 16
_TQ = 256
_TM = 128
_TW = 256
_TA = 512


def _wprep_body(W1, b1, Wq, Wk, Wv, Wg1, Wd2, bd2, bg1,
                Aq, cq, Ak, ck, Av, cv, Wd2g):
    w1 = W1[:]
    g1 = Wg1[:]
    Aq[:] = jnp.dot(jnp.dot(w1, Wq[:], precision=_HI), g1, precision=_HI)
    Ak[:] = jnp.dot(jnp.dot(w1, Wk[:], precision=_HI), g1, precision=_HI)
    Av[:] = jnp.dot(w1, Wv[:], precision=_HI)
    b1v = b1[:]
    cq[:] = (jnp.dot(jnp.dot(b1v, Wq[:], precision=_HI), g1, precision=_HI)
             + jnp.dot(bd2[:], g1, precision=_HI) + bg1[:])
    ck[:] = jnp.dot(jnp.dot(b1v, Wk[:], precision=_HI), g1, precision=_HI)
    cv[:] = jnp.dot(b1v, Wv[:], precision=_HI)
    Wd2g[:] = jnp.dot(Wd2[:], g1, precision=_HI)


def _wprep(W1, b1, Wq, Wk, Wv, Wg1, Wd2, bd2, bg1):
    f32 = jnp.float32
    outs = (
        jax.ShapeDtypeStruct((_TF, _D), f32),
        jax.ShapeDtypeStruct((1, _D), f32),
        jax.ShapeDtypeStruct((_TF, _D), f32),
        jax.ShapeDtypeStruct((1, _D), f32),
        jax.ShapeDtypeStruct((_TF, _D), f32),
        jax.ShapeDtypeStruct((1, _D), f32),
        jax.ShapeDtypeStruct((_D, _D), f32),
    )
    return pl.pallas_call(_wprep_body, out_shape=outs)(
        W1, b1.reshape(1, _D), Wq, Wk, Wv, Wg1, Wd2,
        bd2.reshape(1, _D), bg1.reshape(1, _D))


def _feats_body(xq_ref, W0a, b0a, W0b, b0b, table_ref):
    xq = xq_ref[:]
    f1 = jnp.maximum(jnp.dot(xq, W0a[:]) + b0a[:], 0.0)
    feats = jnp.dot(f1, W0b[:]) + b0b[:]
    table_ref[:] = jnp.concatenate(
        [xq, jnp.zeros((_TA, _INF), jnp.float32), feats], axis=1)


def _feats(xyzf, W0a, b0a, W0b, b0b):
    grid = (_B * _N // _TA,)
    row = lambda t: (t, 0)
    full = lambda t: (0, 0)
    return pl.pallas_call(
        _feats_body, grid=grid,
        in_specs=[
            pl.BlockSpec((_TA, _INF), row),
            pl.BlockSpec((_INF, _TF), full),
            pl.BlockSpec((1, _TF), full),
            pl.BlockSpec((_TF, _TF), full),
            pl.BlockSpec((1, _TF), full),
        ],
        out_specs=pl.BlockSpec((_TA, _TW), row),
        out_shape=jax.ShapeDtypeStruct((_B * _N, _TW), jnp.float32),
    )(xyzf, W0a, b0a.reshape(1, _TF), W0b, b0b.reshape(1, _TF))


def _topk_body(xq_ref, xf_ref, idx_ref, *, batch):
    xq = xq_ref[0]
    xf = xf_ref[0]

    d = -2.0 * lax.dot_general(xq, xf, (((1,), (1,)), ((), ())))
    d = d + jnp.sum(xq * xq, axis=1, keepdims=True)
    d = d + jnp.sum(xf * xf, axis=1)[None, :]

    colf = lax.broadcasted_iota(jnp.int32, (_TQ, _N), 1).astype(jnp.float32)
    big = jnp.float32(3.0e38)
    vals = d
    sels = []
    for _ in range(_K):
        m = jnp.min(vals, axis=1, keepdims=True)
        cand = jnp.where(vals <= m, colf, jnp.float32(_N))
        sel = jnp.min(cand, axis=1, keepdims=True)
        sels.append(sel)
        vals = jnp.where(colf == sel, big, vals)
    idx_ref[:] = jnp.concatenate(sels, axis=1).astype(jnp.int32) + batch * _N


def _topk(batch, xyz):
    nt = _N // _TQ
    grid = (nt,)
    return pl.pallas_call(
        functools.partial(_topk_body, batch=batch),
        grid=grid,
        in_specs=[
            pl.BlockSpec((1, _TQ, _INF), lambda t: (batch, t, 0)),
            pl.BlockSpec((1, _N, _INF), lambda t: (batch, 0, 0)),
        ],
        out_specs=pl.BlockSpec((_TQ, _K), lambda t: (t, 0)),
        out_shape=jax.ShapeDtypeStruct((_N, _K), jnp.int32),
    )(xyz, xyz)


def _sc_gather(table, idx_flat):
    tot = idx_flat.shape[0]
    nw = 32
    per_w = tot // nw
    ch = 128
    n_ch = per_w // ch

    mesh = plsc.VectorSubcoreMesh(core_axis_name="c", subcore_axis_name="s")

    @functools.partial(
        pl.kernel, mesh=mesh,
        out_type=jax.ShapeDtypeStruct((tot, _TW), jnp.float32),
        scratch_types=[
            pltpu.VMEM((per_w,), jnp.int32),
            pltpu.VMEM((ch, _TW), jnp.float32),
            pltpu.VMEM((ch, _TW), jnp.float32),
            pltpu.SemaphoreType.DMA,
            pltpu.SemaphoreType.DMA,
        ],
    )
    def gather_kernel(table_hbm, idx_hbm, out_hbm, idx_v, rows_a, rows_b,
                      sem_a, sem_b):
        wid = lax.axis_index("s") * 2 + lax.axis_index("c")
        base = wid * per_w
        pltpu.sync_copy(idx_hbm.at[pl.ds(base, per_w)], idx_v)
        bufs = (rows_a, rows_b)
        sems = (sem_a, sem_b)
        cps = [None, None]
        for c in range(n_ch + 1):
            if c < n_ch:
                p = c % 2
                cps[p] = pltpu.async_copy(
                    table_hbm.at[idx_v.at[pl.ds(c * ch, ch)]],
                    bufs[p], sems[p])
            if c >= 1:
                p = (c - 1) % 2
                cps[p].wait()
                pltpu.sync_copy(bufs[p],
                                out_hbm.at[pl.ds(base + (c - 1) * ch, ch)])

    return gather_kernel(table, idx_flat)


def _stage3_body(g_ref, xyz_ref, pre_ref,
                 Aq, cq, Ak, ck, Av, cv,
                 Wd1, bd1, Wd2, bd2, Wd2g, Wg2, bg2, W2, b2,
                 attn_ref, res_ref):
    g = g_ref[:]
    xq = xyz_ref[:]
    fq = pre_ref[:]
    pos = xq[:, None, :] - g[:, :, 0:_INF]
    pos2 = pos.reshape(_TM * _K, _INF)
    h = jnp.maximum(jnp.dot(pos2, Wd1[:]) + bd1[:], 0.0)
    pe = jnp.dot(h, Wd2[:]) + bd2[:]
    a3 = jnp.dot(h, Wd2g[:])

    f2 = g[:, :, _TF:].reshape(_TM * _K, _TF)
    kg2 = jnp.dot(f2, Ak[:]) + ck[:]
    v2 = jnp.dot(f2, Av[:]) + cv[:]
    qg = jnp.dot(fq, Aq[:]) + cq[:]
    qg2 = jnp.broadcast_to(qg[:, None, :], (_TM, _K, _D))
    qg2 = qg2.reshape(_TM * _K, _D)

    l1 = jnp.maximum(qg2 - kg2 + a3, 0.0)
    logits = (jnp.dot(l1, Wg2[:]) + bg2[:]) * jnp.float32(1.0 / 16.0)

    lg3 = logits.reshape(_TM, _K, _D)
    m = jnp.max(lg3, axis=1, keepdims=True)
    e = jnp.exp(lg3 - m)
    s = jnp.sum(e, axis=1, keepdims=True)
    attn = e / s
    attn_ref[:] = attn

    pe3 = pe.reshape(_TM, _K, _D)
    v3 = v2.reshape(_TM, _K, _D)
    wsum = jnp.sum(attn * (v3 + pe3), axis=1)
    res_ref[:] = jnp.dot(wsum, W2[:]) + b2[:] + fq


def _stage3_body_alias(attn_in, res_in, *args):
    _stage3_body(*args)


def _stage3(batch, g3, xyzf, table, wp, prev):
    f32 = jnp.float32
    (Aq, cq, Ak, ck, Av, cv, Wd2g) = wp[0]
    (Wd1, bd1, Wd2, bd2, Wg2, bg2, W2, b2) = wp[1]
    nt = _N // _TM
    grid = (nt,)
    row = lambda t: (batch * nt + t, 0)
    row3 = lambda t: (batch * nt + t, 0, 0)
    grow = lambda t: (t, 0, 0)
    row_f = lambda t: (batch * nt + t, 1)
    full = lambda t: (0, 0)
    in_specs = [
        pl.BlockSpec((_TM, _K, _TW), grow),
        pl.BlockSpec((_TM, _INF), row),
        pl.BlockSpec((_TM, _TF), row_f),
        pl.BlockSpec((_TF, _D), full),
        pl.BlockSpec((1, _D), full),
        pl.BlockSpec((_TF, _D), full),
        pl.BlockSpec((1, _D), full),
        pl.BlockSpec((_TF, _D), full),
        pl.BlockSpec((1, _D), full),
        pl.BlockSpec((_INF, _D), full),
        pl.BlockSpec((1, _D), full),
        pl.BlockSpec((_D, _D), full),
        pl.BlockSpec((1, _D), full),
        pl.BlockSpec((_D, _D), full),
        pl.BlockSpec((_D, _D), full),
        pl.BlockSpec((1, _D), full),
        pl.BlockSpec((_D, _TF), full),
        pl.BlockSpec((1, _TF), full),
    ]
    out_specs = [
        pl.BlockSpec((_TM, _K, _D), row3),
        pl.BlockSpec((_TM, _TF), row),
    ]
    outs = (
        jax.ShapeDtypeStruct((_B * _N, _K, _D), f32),
        jax.ShapeDtypeStruct((_B * _N, _TF), f32),
    )
    operands = (g3, xyzf, table, Aq, cq, Ak, ck, Av, cv,
                Wd1, bd1.reshape(1, _D), Wd2, bd2.reshape(1, _D), Wd2g,
                Wg2, bg2.reshape(1, _D), W2, b2.reshape(1, _TF))
    if prev is None:
        return pl.pallas_call(
            _stage3_body, grid=grid, in_specs=in_specs,
            out_specs=out_specs, out_shape=outs)(*operands)
    attn_prev, res_prev = prev
    in_specs = [pl.BlockSpec(memory_space=pl.ANY),
                pl.BlockSpec(memory_space=pl.ANY)] + in_specs
    return pl.pallas_call(
        _stage3_body_alias, grid=grid, in_specs=in_specs,
        out_specs=out_specs, out_shape=outs,
        input_output_aliases={0: 0, 1: 1},
    )(attn_prev, res_prev, *operands)


def kernel(xyz, W0a, b0a, W0b, b0b, W1, b1, W2, b2, Wd1, bd1, Wd2, bd2,
           Wg1, bg1, Wg2, bg2, Wq, Wk, Wv):
    assert xyz.shape == (_B, _N, _INF)
    wp0 = _wprep(W1, b1, Wq, Wk, Wv, Wg1, Wd2, bd2, bg1)
    wp = (wp0, (Wd1, bd1, Wd2, bd2, Wg2, bg2, W2, b2))
    xyzf = xyz.reshape(_B * _N, _INF)
    table = _feats(xyzf, W0a, b0a, W0b, b0b)
    prev = None
    for b in range(_B):
        idx = _topk(b, xyz)
        g = _sc_gather(table, idx.reshape(_N * _K))
        prev = _stage3(b, g.reshape(_N, _K, _TW), xyzf, table, wp, prev)
    attn, res = prev[0], prev[1]
    return (res.reshape(_B, _N, _TF), attn.reshape(_B, _N, _K, _D))
